# Initial kernel scaffold; baseline (speedup 1.0000x reference)
import jax
import jax.numpy as jnp
from jax.experimental import pallas as pl


def kernel(x, slot_keys, Wk, bk, Wq, bq, Wv, bv, scale, ln_g, ln_b, Wo, bo):
    raise NotImplementedError("write your pallas kernel here")


# speedup vs baseline: 1.7729x; 1.7729x over previous
import functools

import jax
import jax.numpy as jnp
from jax.experimental import pallas as pl
from jax.experimental.pallas import tpu as pltpu

DIM = 512
NUM_SLOTS = 64
CHUNK = 64
EPS = 1e-5
TOK = 512


def _softmax(logits):
    m = jnp.max(logits, axis=-1, keepdims=True)
    e = jnp.exp(logits - m)
    return e / jnp.sum(e, axis=-1, keepdims=True)


def _fused_kernel(x_ref, sk_ref, wk_ref, bk_ref, wq_ref, bq_ref, wv_ref,
                  bv_ref, scale_ref, g_ref, b_ref, wo_ref, bo_ref, o_ref):
    x = x_ref[...]
    k = jnp.dot(x, wk_ref[...], preferred_element_type=jnp.float32) + bk_ref[...]
    q = jnp.dot(x, wq_ref[...], preferred_element_type=jnp.float32) + bq_ref[...]
    v = jnp.dot(x, wv_ref[...], preferred_element_type=jnp.float32) + bv_ref[...]

    sk = sk_ref[...]
    scale = scale_ref[0, 0]
    dn = (((1,), (1,)), ((), ()))
    ww = _softmax(jax.lax.dot_general(k, sk, dn,
                                      preferred_element_type=jnp.float32) * scale)
    rw = _softmax(jax.lax.dot_general(q, sk, dn,
                                      preferred_element_type=jnp.float32) * scale)

    a = jax.lax.dot_general(rw, ww, dn, preferred_element_type=jnp.float32)
    r = jax.lax.broadcasted_iota(jnp.int32, (TOK, TOK), 0)
    u = jax.lax.broadcasted_iota(jnp.int32, (TOK, TOK), 1)
    mask = (r // CHUNK == u // CHUNK) & (u <= r)
    a = jnp.where(mask, a, 0.0)
    ret = jnp.dot(a, v, preferred_element_type=jnp.float32)

    mu = jnp.mean(ret, axis=-1, keepdims=True)
    cen = ret - mu
    var = jnp.mean(cen * cen, axis=-1, keepdims=True)
    ln = cen * jax.lax.rsqrt(var + EPS) * g_ref[...] + b_ref[...]
    out = jnp.dot(ln, wo_ref[...], preferred_element_type=jnp.float32) + bo_ref[...]
    o_ref[...] = x + out


def kernel(x, slot_keys, Wk, bk, Wq, bq, Wv, bv, scale, ln_g, ln_b, Wo, bo):
    b, s, d = x.shape
    n = b * s
    x2 = x.reshape(n, d)
    full = lambda i: (0, 0)
    wspec = pl.BlockSpec((d, d), full)
    vspec = pl.BlockSpec((1, d), full)
    out = pl.pallas_call(
        _fused_kernel,
        out_shape=jax.ShapeDtypeStruct((n, d), x.dtype),
        grid=(n // TOK,),
        in_specs=[
            pl.BlockSpec((TOK, d), lambda i: (i, 0)),
            pl.BlockSpec((NUM_SLOTS, d), full),
            wspec, vspec,
            wspec, vspec,
            wspec, vspec,
            pl.BlockSpec((1, 1), full, memory_space=pltpu.SMEM),
            vspec, vspec,
            wspec, vspec,
        ],
        out_specs=pl.BlockSpec((TOK, d), lambda i: (i, 0)),
        compiler_params=pltpu.CompilerParams(
            dimension_semantics=("parallel",),
        ),
        name="slot_memory_phasor",
    )(x2, slot_keys, Wk, bk.reshape(1, d), Wq, bq.reshape(1, d),
      Wv, bv.reshape(1, d), scale.reshape(1, 1), ln_g.reshape(1, d),
      ln_b.reshape(1, d), Wo, bo.reshape(1, d))
    return out.reshape(b, s, d)
